# baseline (speedup 1.0000x reference)
import functools

import jax
import jax.numpy as jnp
from jax import lax
from jax.experimental import pallas as pl
from jax.experimental.pallas import tpu as pltpu
from jax.experimental.pallas import tpu_sc as plsc

NUM_CORES = 2
NUM_SUBCORES = 16
NUM_WORKERS = NUM_CORES * NUM_SUBCORES
EMBED = 32


def _body(table_hbm, idx_hbm, out_hbm, idx_v, rows_v, sem_i, sem_g, sem_o):
    batch, seq = out_hbm.shape[0], out_hbm.shape[1]
    rows_per_w = batch // NUM_WORKERS
    wid = lax.axis_index("s") * NUM_CORES + lax.axis_index("c")
    base = wid * rows_per_w

    def idx_copy(j, b):
        return pltpu.make_async_copy(
            idx_hbm.at[pl.ds((base + j) * seq, seq)], idx_v.at[b],
            sem_i.at[b])

    def gather_copy(b):
        return pltpu.make_async_copy(table_hbm.at[idx_v.at[b]],
                                     rows_v.at[b], sem_g.at[b])

    def out_copy(j, b):
        return pltpu.make_async_copy(
            rows_v.at[b], out_hbm.at[base + j], sem_o.at[b])

    idx_copy(0, 0).start()
    idx_copy(1, 1).start()
    idx_copy(0, 0).wait()
    gather_copy(0).start()

    def step(jo, carry):
        for b in range(2):
            j = jo * 2 + b
            o = 1 - b

            @pl.when(j >= 1)
            def _():
                out_copy(j - 1, o).wait()

            @pl.when(j + 1 < rows_per_w)
            def _():
                idx_copy(j + 1, o).wait()
                gather_copy(o).start()

            gather_copy(b).wait()

            @pl.when(j + 2 < rows_per_w)
            def _():
                idx_copy(j + 2, b).start()

            out_copy(j, b).start()
        return carry

    lax.fori_loop(0, rows_per_w // 2, step, 0)
    out_copy(rows_per_w - 1, (rows_per_w - 1) % 2).wait()


@functools.partial(jax.jit, static_argnames=("batch", "seq"))
def _gather(table, idx, batch, seq):
    mesh = plsc.VectorSubcoreMesh(
        core_axis_name="c", subcore_axis_name="s",
        num_cores=NUM_CORES, num_subcores=NUM_SUBCORES)
    return pl.kernel(
        _body,
        out_type=jax.ShapeDtypeStruct((batch, seq, EMBED), jnp.float32),
        mesh=mesh,
        scratch_types=[
            pltpu.VMEM((2, 200), jnp.int32),
            pltpu.VMEM((2, 200, EMBED), jnp.float32),
            pltpu.SemaphoreType.DMA((2,)),
            pltpu.SemaphoreType.DMA((2,)),
            pltpu.SemaphoreType.DMA((2,)),
        ],
        compiler_params=pltpu.CompilerParams(use_tc_tiling_on_sc=False),
    )(table, idx)


def kernel(regimes, table):
    b, t = regimes.shape
    idx = regimes.reshape(-1).astype(jnp.int32)
    return _gather(table, idx, b, t)
